# baseline (speedup 1.0000x reference)
import functools

import jax
import jax.numpy as jnp
from jax import lax
from jax.experimental import pallas as pl
from jax.experimental.pallas import tpu as pltpu
from jax.experimental.pallas import tpu_sc as plsc

B2 = 32
BH = 16
NS = [10242, 2562, 642, 162]
NP = [10368, 2688, 768, 256]
NW = 32
EPS = 1e-5


def _sc_gather(table, idx2d):
    R, D = table.shape
    M = idx2d.shape[0] * 128
    mw = M // NW
    grp = max(1, 128 // D)
    RB = grp * 128
    nfull = mw // RB
    grem = (mw % RB) // 128

    mesh = plsc.VectorSubcoreMesh(core_axis_name="c", subcore_axis_name="s")

    @functools.partial(
        pl.kernel,
        mesh=mesh,
        compiler_params=pltpu.CompilerParams(use_tc_tiling_on_sc=False),
        out_type=jax.ShapeDtypeStruct((M, D), jnp.float32),
        scratch_types=[
            pltpu.VMEM((RB // 128, 1, 128), jnp.int32),
            pltpu.VMEM((RB, D), jnp.float32),
            pltpu.SemaphoreType.DMA,
        ],
    )
    def k(table_hbm, idx_hbm, out_hbm, idx_v, rows_v, sem):
        wid = lax.axis_index("s") * 2 + lax.axis_index("c")
        row0 = wid * (mw // 128)

        def do_block(brow, g):
            pltpu.sync_copy(idx_hbm.at[pl.ds(brow, g)], idx_v.at[pl.ds(0, g)])
            descs = [
                pltpu.async_copy(
                    table_hbm.at[idx_v.at[j, 0]],
                    rows_v.at[pl.ds(j * 128, 128)],
                    sem,
                )
                for j in range(g)
            ]
            for d in descs:
                d.wait()
            pltpu.sync_copy(
                rows_v.at[pl.ds(0, g * 128)],
                out_hbm.at[pl.ds(brow * 128, g * 128)],
            )

        if nfull:
            def body(i, carry):
                do_block(row0 + i * grp, grp)
                return carry
            lax.fori_loop(0, nfull, body, 0)
        if grem:
            do_block(row0 + nfull * grp, grem)

    return k(table, idx2d)


def _gather_rows(x, nodes, np_dst):
    _, np_src, C = x.shape
    nodes = jnp.pad(nodes, (0, 7 * np_dst - nodes.shape[0]))
    idx = (jnp.arange(B2, dtype=jnp.int32) * np_src)[:, None] + nodes[None, :]
    idx2d = idx.reshape(-1, 1, 128)
    return _sc_gather(x.reshape(B2 * np_src, C), idx2d)


def _conv_tc(G, W, sc2, sh2, apply_f, n_real):
    _, NPd, KC = G.shape
    Cout = W.shape[1]
    TN = 128
    NT = NPd // TN

    def kern(g_ref, w_ref, sc_ref, sh_ref, y_ref, st_ref):
        b = pl.program_id(0)
        t = pl.program_id(1)
        g = g_ref[0]
        if apply_f:
            is0 = b < BH
            sc = jnp.where(is0, sc_ref[0:1, :], sc_ref[1:2, :])
            sh = jnp.where(is0, sh_ref[0:1, :], sh_ref[1:2, :])
            a = g * sc + sh
            g = jnp.maximum(a, 0.2 * a)
        y = jnp.dot(g, w_ref[...], preferred_element_type=jnp.float32)
        y_ref[0] = y
        rows = lax.broadcasted_iota(jnp.int32, (TN, Cout), 0) + t * TN
        ym = jnp.where(rows < n_real, y, 0.0)
        s1 = jnp.sum(ym, axis=0, keepdims=True)
        s2 = jnp.sum(ym * ym, axis=0, keepdims=True)
        z = jnp.zeros((1, Cout), jnp.float32)
        contrib = jnp.where(
            b < BH,
            jnp.concatenate([s1, s2, z, z], axis=0),
            jnp.concatenate([z, z, s1, s2], axis=0),
        )

        @pl.when(jnp.logical_and(b == 0, t == 0))
        def _():
            st_ref[...] = jnp.zeros((4, Cout), jnp.float32)

        st_ref[...] += contrib

    return pl.pallas_call(
        kern,
        grid=(B2, NT),
        in_specs=[
            pl.BlockSpec((1, TN, KC), lambda b, t: (b, t, 0)),
            pl.BlockSpec((KC, Cout), lambda b, t: (0, 0)),
            pl.BlockSpec((2, KC), lambda b, t: (0, 0)),
            pl.BlockSpec((2, KC), lambda b, t: (0, 0)),
        ],
        out_specs=[
            pl.BlockSpec((1, TN, Cout), lambda b, t: (b, t, 0)),
            pl.BlockSpec((4, Cout), lambda b, t: (0, 0)),
        ],
        out_shape=[
            jax.ShapeDtypeStruct((B2, NPd, Cout), jnp.float32),
            jax.ShapeDtypeStruct((4, Cout), jnp.float32),
        ],
    )(G, W, sc2, sh2)


def _pool_tc(Gp, sc2, sh2):
    _, NPd, _, C = Gp.shape
    TN = 128
    NT = NPd // TN

    def kern(g_ref, sc_ref, sh_ref, o_ref):
        b = pl.program_id(0)
        is0 = b < BH
        sc = jnp.where(is0, sc_ref[0:1, :], sc_ref[1:2, :])
        sh = jnp.where(is0, sh_ref[0:1, :], sh_ref[1:2, :])
        acc = jnp.zeros((TN, C), jnp.float32)
        for k in range(7):
            a = g_ref[0, :, k, :] * sc + sh
            acc += jnp.maximum(a, 0.2 * a)
        o_ref[0] = acc * (1.0 / 7.0)

    return pl.pallas_call(
        kern,
        grid=(B2, NT),
        in_specs=[
            pl.BlockSpec((1, TN, 7, C), lambda b, t: (b, t, 0, 0)),
            pl.BlockSpec((2, C), lambda b, t: (0, 0)),
            pl.BlockSpec((2, C), lambda b, t: (0, 0)),
        ],
        out_specs=pl.BlockSpec((1, TN, C), lambda b, t: (b, t, 0)),
        out_shape=jax.ShapeDtypeStruct((B2, NPd, C), jnp.float32),
    )(Gp, sc2, sh2)


def _act_tc(y, sc2, sh2):
    _, NPd, C = y.shape
    TN = 128
    NT = NPd // TN

    def kern(y_ref, sc_ref, sh_ref, o_ref):
        b = pl.program_id(0)
        is0 = b < BH
        sc = jnp.where(is0, sc_ref[0:1, :], sc_ref[1:2, :])
        sh = jnp.where(is0, sh_ref[0:1, :], sh_ref[1:2, :])
        a = y_ref[0] * sc + sh
        o_ref[0] = jnp.maximum(a, 0.2 * a)

    return pl.pallas_call(
        kern,
        grid=(B2, NT),
        in_specs=[
            pl.BlockSpec((1, TN, C), lambda b, t: (b, t, 0)),
            pl.BlockSpec((2, C), lambda b, t: (0, 0)),
            pl.BlockSpec((2, C), lambda b, t: (0, 0)),
        ],
        out_specs=pl.BlockSpec((1, TN, C), lambda b, t: (b, t, 0)),
        out_shape=jax.ShapeDtypeStruct((B2, NPd, C), jnp.float32),
    )(y, sc2, sh2)


def _bn_params(st, n_real, g, be):
    cnt = float(BH * n_real)
    m = jnp.stack([st[0], st[2]]) / cnt
    ex2 = jnp.stack([st[1], st[3]]) / cnt
    v = ex2 - m * m
    sc = g[None, :] * lax.rsqrt(v + EPS)
    sh = be[None, :] - m * sc
    return sc, sh


def _conv_layer(x, no, lvl_dst, W, sc2, sh2, apply_f):
    np_dst = NP[lvl_dst]
    C = x.shape[2]
    rows = _gather_rows(x, no, np_dst)
    G = rows.reshape(B2, np_dst, 7 * C)
    return _conv_tc(G, W, sc2, sh2, apply_f, NS[lvl_dst])


def _pool_layer(x, no_src, lvl_dst, sc2, sh2):
    np_dst = NP[lvl_dst]
    C = x.shape[2]
    n2 = NS[lvl_dst]
    rows = _gather_rows(x, no_src[: 7 * n2], np_dst)
    Gp = rows.reshape(B2, np_dst, 7, C)
    return _pool_tc(Gp, sc2, sh2)


def kernel(x_l, x_r, no0, no1, no2, no3,
           W0, W1, W2, W3, W4, W5, W6, W7, W8,
           b0, b1, b2, b3, b4, b5, b6, b7, b8,
           g0, g1, g2, g3, g4, g5, g6, g7, g8,
           be0, be1, be2, be3, be4, be5, be6, be7, be8):
    x = jnp.concatenate([x_l, x_r], axis=0)
    x = jnp.pad(x, ((0, 0), (0, NP[0] - NS[0]), (0, 13)))
    W0p = jnp.pad(W0.reshape(7, 3, 32), ((0, 0), (0, 13), (0, 0)))
    W0p = W0p.reshape(112, 32)

    def t7(a):
        return jnp.tile(a, (1, 7))

    def dz(KC):
        z = jnp.zeros((2, KC), jnp.float32)
        return z, z

    y, st = _conv_layer(x, no0, 0, W0p, *dz(112), False)
    sc, sh = _bn_params(st, NS[0], g0, be0)
    y, st = _conv_layer(y, no0, 0, W1, t7(sc), t7(sh), True)
    sc, sh = _bn_params(st, NS[0], g1, be1)
    y, st = _conv_layer(y, no0, 0, W2, t7(sc), t7(sh), True)
    sc, sh = _bn_params(st, NS[0], g2, be2)
    y = _pool_layer(y, no0, 1, sc, sh)
    y, st = _conv_layer(y, no1, 1, W3, *dz(224), False)
    sc, sh = _bn_params(st, NS[1], g3, be3)
    y, st = _conv_layer(y, no1, 1, W4, t7(sc), t7(sh), True)
    sc, sh = _bn_params(st, NS[1], g4, be4)
    y = _pool_layer(y, no1, 2, sc, sh)
    y, st = _conv_layer(y, no2, 2, W5, *dz(448), False)
    sc, sh = _bn_params(st, NS[2], g5, be5)
    y, st = _conv_layer(y, no2, 2, W6, t7(sc), t7(sh), True)
    sc, sh = _bn_params(st, NS[2], g6, be6)
    y = _pool_layer(y, no2, 3, sc, sh)
    y, st = _conv_layer(y, no3, 3, W7, *dz(896), False)
    sc, sh = _bn_params(st, NS[3], g7, be7)
    y, st = _conv_layer(y, no3, 3, W8, t7(sc), t7(sh), True)
    sc, sh = _bn_params(st, NS[3], g8, be8)
    y = _act_tc(y, sc, sh)

    out = y[:, : NS[3], :]
    out_l, out_r = out[:BH], out[BH:]
    return (out_l, out_l, out_l, out_l, out_r, out_r, out_r, out_r)


# speedup vs baseline: 1.6984x; 1.6984x over previous
import functools

import jax
import jax.numpy as jnp
from jax import lax
from jax.experimental import pallas as pl
from jax.experimental.pallas import tpu as pltpu
from jax.experimental.pallas import tpu_sc as plsc

B2 = 32
BH = 16
NS = [10242, 2562, 642, 162]
NP = [10368, 2688, 768, 256]
NW = 32
EPS = 1e-5


def _sc_gather(table, idx2d):
    R, D = table.shape
    M = idx2d.shape[0] * 128
    mw = M // NW
    grp = max(1, 128 // D)
    RB = grp * 128
    nfull = mw // RB
    grem = (mw % RB) // 128

    mesh = plsc.VectorSubcoreMesh(core_axis_name="c", subcore_axis_name="s")

    @functools.partial(
        pl.kernel,
        mesh=mesh,
        compiler_params=pltpu.CompilerParams(use_tc_tiling_on_sc=False),
        out_type=jax.ShapeDtypeStruct((M, D), jnp.float32),
        scratch_types=[
            pltpu.VMEM((RB // 128, 1, 128), jnp.int32),
            pltpu.VMEM((RB, D), jnp.float32),
            pltpu.SemaphoreType.DMA,
        ],
    )
    def k(table_hbm, idx_hbm, out_hbm, idx_v, rows_v, sem):
        wid = lax.axis_index("s") * 2 + lax.axis_index("c")
        row0 = wid * (mw // 128)

        def do_block(brow, g):
            pltpu.sync_copy(idx_hbm.at[pl.ds(brow, g)], idx_v.at[pl.ds(0, g)])
            descs = [
                pltpu.async_copy(
                    table_hbm.at[idx_v.at[j, 0]],
                    rows_v.at[pl.ds(j * 128, 128)],
                    sem,
                )
                for j in range(g)
            ]
            for d in descs:
                d.wait()
            pltpu.sync_copy(
                rows_v.at[pl.ds(0, g * 128)],
                out_hbm.at[pl.ds(brow * 128, g * 128)],
            )

        if nfull:
            def body(i, carry):
                do_block(row0 + i * grp, grp)
                return carry
            lax.fori_loop(0, nfull, body, 0)
        if grem:
            do_block(row0 + nfull * grp, grem)

    return k(table, idx2d)


def _gather_rows(x, nodes, np_dst):
    _, np_src, C = x.shape
    nodes = jnp.pad(nodes, (0, 7 * np_dst - nodes.shape[0]))
    idx = (jnp.arange(B2, dtype=jnp.int32) * np_src)[:, None] + nodes[None, :]
    idx2d = idx.reshape(-1, 1, 128)
    return _sc_gather(x.reshape(B2 * np_src, C), idx2d)


def _conv_tc(G, W, sc2, sh2, apply_f, n_real):
    _, NPd, KC = G.shape
    Cout = W.shape[1]
    R2 = B2 * NPd
    TN = 1024 if KC <= 512 else 512
    TN = min(TN, R2)
    NT = R2 // TN
    Gf = G.reshape(R2, KC)

    def kern(g_ref, w_ref, sc_ref, sh_ref, y_ref, st_ref):
        t = pl.program_id(0)
        g = g_ref[...]
        is0 = t * TN < BH * NPd
        if apply_f:
            sc = jnp.where(is0, sc_ref[0:1, :], sc_ref[1:2, :])
            sh = jnp.where(is0, sh_ref[0:1, :], sh_ref[1:2, :])
            a = g * sc + sh
            g = jnp.maximum(a, 0.2 * a)
        y = jnp.dot(g, w_ref[...], preferred_element_type=jnp.float32)
        y_ref[...] = y
        rows = lax.broadcasted_iota(jnp.int32, (TN, Cout), 0) + t * TN
        node = lax.rem(rows, NPd)
        ym = jnp.where(node < n_real, y, 0.0)
        s1 = jnp.sum(ym, axis=0, keepdims=True)
        s2 = jnp.sum(ym * ym, axis=0, keepdims=True)
        z = jnp.zeros((2, Cout), jnp.float32)
        s12 = jnp.concatenate([s1, s2], axis=0)
        contrib = jnp.where(
            is0,
            jnp.concatenate([s12, z, z, z], axis=0),
            jnp.concatenate([z, s12, z, z], axis=0),
        )

        @pl.when(t == 0)
        def _():
            st_ref[...] = jnp.zeros((8, Cout), jnp.float32)

        st_ref[...] += contrib

    y, st = pl.pallas_call(
        kern,
        grid=(NT,),
        in_specs=[
            pl.BlockSpec((TN, KC), lambda t: (t, 0)),
            pl.BlockSpec((KC, Cout), lambda t: (0, 0)),
            pl.BlockSpec((2, KC), lambda t: (0, 0)),
            pl.BlockSpec((2, KC), lambda t: (0, 0)),
        ],
        out_specs=[
            pl.BlockSpec((TN, Cout), lambda t: (t, 0)),
            pl.BlockSpec((8, Cout), lambda t: (0, 0)),
        ],
        out_shape=[
            jax.ShapeDtypeStruct((R2, Cout), jnp.float32),
            jax.ShapeDtypeStruct((8, Cout), jnp.float32),
        ],
    )(Gf, W, sc2, sh2)
    return y.reshape(B2, NPd, Cout), st


def _pool_tc(Gp, sc2, sh2):
    _, NPd, _, C = Gp.shape
    R2 = B2 * NPd
    TN = min(1024, R2)
    NT = R2 // TN
    Gf = Gp.reshape(R2, 7, C)

    def kern(g_ref, sc_ref, sh_ref, o_ref):
        t = pl.program_id(0)
        is0 = t * TN < BH * NPd
        sc = jnp.where(is0, sc_ref[0:1, :], sc_ref[1:2, :])
        sh = jnp.where(is0, sh_ref[0:1, :], sh_ref[1:2, :])
        acc = jnp.zeros((TN, C), jnp.float32)
        for k in range(7):
            a = g_ref[:, k, :] * sc + sh
            acc += jnp.maximum(a, 0.2 * a)
        o_ref[...] = acc * (1.0 / 7.0)

    out = pl.pallas_call(
        kern,
        grid=(NT,),
        in_specs=[
            pl.BlockSpec((TN, 7, C), lambda t: (t, 0, 0)),
            pl.BlockSpec((2, C), lambda t: (0, 0)),
            pl.BlockSpec((2, C), lambda t: (0, 0)),
        ],
        out_specs=pl.BlockSpec((TN, C), lambda t: (t, 0)),
        out_shape=jax.ShapeDtypeStruct((R2, C), jnp.float32),
    )(Gf, sc2, sh2)
    return out.reshape(B2, NPd, C)


def _act_tc(y, sc2, sh2):
    _, NPd, C = y.shape
    R2 = B2 * NPd
    TN = min(1024, R2)
    NT = R2 // TN
    yf = y.reshape(R2, C)

    def kern(y_ref, sc_ref, sh_ref, o_ref):
        t = pl.program_id(0)
        is0 = t * TN < BH * NPd
        sc = jnp.where(is0, sc_ref[0:1, :], sc_ref[1:2, :])
        sh = jnp.where(is0, sh_ref[0:1, :], sh_ref[1:2, :])
        a = y_ref[...] * sc + sh
        o_ref[...] = jnp.maximum(a, 0.2 * a)

    out = pl.pallas_call(
        kern,
        grid=(NT,),
        in_specs=[
            pl.BlockSpec((TN, C), lambda t: (t, 0)),
            pl.BlockSpec((2, C), lambda t: (0, 0)),
            pl.BlockSpec((2, C), lambda t: (0, 0)),
        ],
        out_specs=pl.BlockSpec((TN, C), lambda t: (t, 0)),
        out_shape=jax.ShapeDtypeStruct((R2, C), jnp.float32),
    )(yf, sc2, sh2)
    return out.reshape(B2, NPd, C)


def _bn_params(st, n_real, g, be):
    cnt = float(BH * n_real)
    m = jnp.stack([st[0], st[2]]) / cnt
    ex2 = jnp.stack([st[1], st[3]]) / cnt
    v = ex2 - m * m
    sc = g[None, :] * lax.rsqrt(v + EPS)
    sh = be[None, :] - m * sc
    return sc, sh


def _conv_layer(x, no, lvl_dst, W, sc2, sh2, apply_f):
    np_dst = NP[lvl_dst]
    C = x.shape[2]
    rows = _gather_rows(x, no, np_dst)
    G = rows.reshape(B2, np_dst, 7 * C)
    return _conv_tc(G, W, sc2, sh2, apply_f, NS[lvl_dst])


def _pool_layer(x, no_src, lvl_dst, sc2, sh2):
    np_dst = NP[lvl_dst]
    C = x.shape[2]
    n2 = NS[lvl_dst]
    rows = _gather_rows(x, no_src[: 7 * n2], np_dst)
    Gp = rows.reshape(B2, np_dst, 7, C)
    return _pool_tc(Gp, sc2, sh2)


def kernel(x_l, x_r, no0, no1, no2, no3,
           W0, W1, W2, W3, W4, W5, W6, W7, W8,
           b0, b1, b2, b3, b4, b5, b6, b7, b8,
           g0, g1, g2, g3, g4, g5, g6, g7, g8,
           be0, be1, be2, be3, be4, be5, be6, be7, be8):
    x = jnp.concatenate([x_l, x_r], axis=0)
    x = jnp.pad(x, ((0, 0), (0, NP[0] - NS[0]), (0, 13)))
    W0p = jnp.pad(W0.reshape(7, 3, 32), ((0, 0), (0, 13), (0, 0)))
    W0p = W0p.reshape(112, 32)

    def t7(a):
        return jnp.tile(a, (1, 7))

    def dz(KC):
        z = jnp.zeros((2, KC), jnp.float32)
        return z, z

    y, st = _conv_layer(x, no0, 0, W0p, *dz(112), False)
    sc, sh = _bn_params(st, NS[0], g0, be0)
    y, st = _conv_layer(y, no0, 0, W1, t7(sc), t7(sh), True)
    sc, sh = _bn_params(st, NS[0], g1, be1)
    y, st = _conv_layer(y, no0, 0, W2, t7(sc), t7(sh), True)
    sc, sh = _bn_params(st, NS[0], g2, be2)
    y = _pool_layer(y, no0, 1, sc, sh)
    y, st = _conv_layer(y, no1, 1, W3, *dz(224), False)
    sc, sh = _bn_params(st, NS[1], g3, be3)
    y, st = _conv_layer(y, no1, 1, W4, t7(sc), t7(sh), True)
    sc, sh = _bn_params(st, NS[1], g4, be4)
    y = _pool_layer(y, no1, 2, sc, sh)
    y, st = _conv_layer(y, no2, 2, W5, *dz(448), False)
    sc, sh = _bn_params(st, NS[2], g5, be5)
    y, st = _conv_layer(y, no2, 2, W6, t7(sc), t7(sh), True)
    sc, sh = _bn_params(st, NS[2], g6, be6)
    y = _pool_layer(y, no2, 3, sc, sh)
    y, st = _conv_layer(y, no3, 3, W7, *dz(896), False)
    sc, sh = _bn_params(st, NS[3], g7, be7)
    y, st = _conv_layer(y, no3, 3, W8, t7(sc), t7(sh), True)
    sc, sh = _bn_params(st, NS[3], g8, be8)
    y = _act_tc(y, sc, sh)

    out = y[:, : NS[3], :]
    out_l, out_r = out[:BH], out[BH:]
    return (out_l, out_l, out_l, out_l, out_r, out_r, out_r, out_r)
